# baseline (speedup 1.0000x reference)
import functools

import jax
import jax.numpy as jnp
from jax import lax
from jax.experimental import pallas as pl
from jax.experimental.pallas import tpu as pltpu
from jax.experimental.pallas import tpu_sc as plsc

D_MODEL = 1024
D_FF = 2048
N_EXPERT = 8
TOP_K = 2
SEQ = 2048
M = 256
NBLK_R = (SEQ * TOP_K) // M + N_EXPERT
PAD_R = NBLK_R * M
NBLK_S = SEQ // M
NBLK = NBLK_R + NBLK_S
ROWS = PAD_R + SEQ


def _router_body(x_ref, gw_ref, gb_ref, pos_ref, w_ref, be_ref):
    x = x_ref[...]
    logits = jnp.dot(x, gw_ref[...], preferred_element_type=jnp.float32)
    logits = logits + gb_ref[...]
    mx = jnp.max(logits, axis=1, keepdims=True)
    ex = jnp.exp(logits - mx)
    probs = ex / jnp.sum(ex, axis=1, keepdims=True)

    iota_e = lax.broadcasted_iota(jnp.int32, (SEQ, N_EXPERT), 1)
    m1 = jnp.max(probs, axis=1, keepdims=True)
    i1 = jnp.min(jnp.where(probs == m1, iota_e, N_EXPERT), axis=1, keepdims=True)
    p2 = jnp.where(iota_e == i1, -1.0, probs)
    m2 = jnp.max(p2, axis=1, keepdims=True)
    i2 = jnp.min(jnp.where(p2 == m2, iota_e, N_EXPERT), axis=1, keepdims=True)

    sel1 = (iota_e == i1)
    sel2 = (iota_e == i2)
    cnt = (sel1 | sel2).astype(jnp.float32)

    ir = lax.broadcasted_iota(jnp.int32, (SEQ, SEQ), 0)
    ic = lax.broadcasted_iota(jnp.int32, (SEQ, SEQ), 1)
    lower = (ic < ir).astype(jnp.float32)
    csum = jnp.dot(lower, cnt, preferred_element_type=jnp.float32)

    counts = jnp.sum(cnt, axis=0, keepdims=True)
    padded = jnp.ceil(counts / M) * M
    ie_r = lax.broadcasted_iota(jnp.int32, (N_EXPERT, N_EXPERT), 0)
    ie_c = lax.broadcasted_iota(jnp.int32, (N_EXPERT, N_EXPERT), 1)
    tri = (ie_r < ie_c).astype(jnp.float32)
    off = jnp.dot(padded, tri, preferred_element_type=jnp.float32)
    end = off + padded

    slotf = off + csum
    pos1 = jnp.sum(jnp.where(sel1, slotf, 0.0), axis=1)
    pos2 = jnp.sum(jnp.where(sel2, slotf, 0.0), axis=1)
    pos_ref[0, :] = pos1.astype(jnp.int32)
    pos_ref[1, :] = pos2.astype(jnp.int32)
    w_ref[0, :] = jnp.sum(jnp.where(sel1, probs, 0.0), axis=1)
    w_ref[1, :] = jnp.sum(jnp.where(sel2, probs, 0.0), axis=1)

    bi = lax.broadcasted_iota(jnp.int32, (NBLK_R, N_EXPERT), 0)
    ee = jnp.broadcast_to(end, (NBLK_R, N_EXPERT))
    beh = jnp.sum((bi.astype(jnp.float32) * M >= ee).astype(jnp.int32), axis=1)
    be_ref[...] = jnp.minimum(beh, N_EXPERT - 1)


def _router(x2d, gate_w, gate_b):
    return pl.pallas_call(
        _router_body,
        out_shape=(
            jax.ShapeDtypeStruct((TOP_K, SEQ), jnp.int32),
            jax.ShapeDtypeStruct((TOP_K, SEQ), jnp.float32),
            jax.ShapeDtypeStruct((NBLK_R,), jnp.int32),
        ),
    )(x2d, gate_w, gate_b.reshape(1, N_EXPERT))


_NC, _NS, _L = 2, 16, 16
_NW = _NC * _NS
_RPW = PAD_R // _NW
_GCH = 32
_NCH = _RPW // _GCH


_NBUF = 3


def _dispatch_body(pos_hbm, w_hbm, x_hbm, z_hbm, zf_hbm, xs_hbm, sc_hbm,
                   pos_v, w_v, tok_v, s_v, buf0, buf1, buf2, gsem, wsem):
    wid = lax.axis_index("s") * _NC + lax.axis_index("c")
    with jax.named_scope("disp_table"):
        pltpu.sync_copy(pos_hbm, pos_v)
        pltpu.sync_copy(w_hbm, w_v)
        pltpu.sync_copy(z_hbm, tok_v)
        pltpu.sync_copy(zf_hbm, s_v)

        lane = lax.iota(jnp.int32, _L)

        @plsc.parallel_loop(0, SEQ // _L, unroll=4)
        def scat_step(j):
            base = j * _L
            t = lane + base
            for k in range(TOP_K):
                idx = pos_v[k, pl.ds(base, _L)]
                plsc.store_scatter(tok_v, [idx], t)
                plsc.store_scatter(s_v, [idx], w_v[k, pl.ds(base, _L)])

    with jax.named_scope("disp_gather"):
        base_w = wid * _RPW
        bufs = (buf0, buf1, buf2)
        gcps = [None] * _NCH
        wcps = [None] * _NCH
        for c in range(_NCH):
            b = base_w + c * _GCH
            if c >= _NBUF:
                wcps[c - _NBUF].wait()
            gcps[c] = pltpu.async_copy(
                x_hbm.at[tok_v.at[pl.ds(b, _GCH)]], bufs[c % _NBUF], gsem)
            if c >= 1:
                gcps[c - 1].wait()
                wcps[c - 1] = pltpu.async_copy(
                    bufs[(c - 1) % _NBUF], xs_hbm.at[pl.ds(b - _GCH, _GCH)], wsem)
        gcps[_NCH - 1].wait()
        wcps[_NCH - 1] = pltpu.async_copy(
            bufs[(_NCH - 1) % _NBUF],
            xs_hbm.at[pl.ds(base_w + (_NCH - 1) * _GCH, _GCH)], wsem)
        for c in range(max(0, _NCH - _NBUF), _NCH):
            wcps[c].wait()
        pltpu.sync_copy(s_v.at[pl.ds(base_w, _RPW)], sc_hbm.at[pl.ds(base_w, _RPW)])


def _dispatch(pos, w, x2d):
    mesh = plsc.VectorSubcoreMesh(core_axis_name="c", subcore_axis_name="s")
    f = pl.kernel(
        _dispatch_body,
        compiler_params=pltpu.CompilerParams(needs_layout_passes=False),
        out_type=(
            jax.ShapeDtypeStruct((PAD_R, D_MODEL // 2), jnp.int32),
            jax.ShapeDtypeStruct((PAD_R,), jnp.float32),
        ),
        mesh=mesh,
        scratch_types=[
            pltpu.VMEM((TOP_K, SEQ), jnp.int32),
            pltpu.VMEM((TOP_K, SEQ), jnp.float32),
            pltpu.VMEM((PAD_R,), jnp.int32),
            pltpu.VMEM((PAD_R,), jnp.float32),
            pltpu.VMEM((_GCH, D_MODEL // 2), jnp.int32),
            pltpu.VMEM((_GCH, D_MODEL // 2), jnp.int32),
            pltpu.VMEM((_GCH, D_MODEL // 2), jnp.int32),
            pltpu.SemaphoreType.DMA,
            pltpu.SemaphoreType.DMA,
        ],
    )
    xb = lax.bitcast_convert_type(
        x2d.astype(jnp.bfloat16).reshape(SEQ, D_MODEL // 2, 2), jnp.int32)
    return f(pos, w, xb, jnp.zeros((PAD_R,), jnp.int32),
             jnp.zeros((PAD_R,), jnp.float32))


def _ffn_body(be_ref, xs_ref, sc_ref, w1_ref, b1_ref, w2_ref, b2_ref, ys_ref):
    xb = xs_ref[...].astype(jnp.float32)
    h = jnp.dot(xb, w1_ref[0], preferred_element_type=jnp.float32)
    h = h + b1_ref[0]
    h = 0.5 * h * (1.0 + lax.erf(h * 0.7071067811865476))
    s = sc_ref[0, 0, :][:, None]
    h = h * s
    y = jnp.dot(h, w2_ref[0], preferred_element_type=jnp.float32)
    ys_ref[...] = y + s * b2_ref[0]


def _ffn(be, xs, scale, w1, b1, w2, b2):
    scale3 = scale.reshape(NBLK_R, 1, M)
    grid_spec = pltpu.PrefetchScalarGridSpec(
        num_scalar_prefetch=1,
        grid=(NBLK_R,),
        in_specs=[
            pl.BlockSpec((M, D_MODEL), lambda b, be: (b, 0)),
            pl.BlockSpec((1, 1, M), lambda b, be: (b, 0, 0)),
            pl.BlockSpec((1, D_MODEL, D_FF), lambda b, be: (be[b], 0, 0)),
            pl.BlockSpec((1, 1, D_FF), lambda b, be: (be[b], 0, 0)),
            pl.BlockSpec((1, D_FF, D_MODEL), lambda b, be: (be[b], 0, 0)),
            pl.BlockSpec((1, 1, D_MODEL), lambda b, be: (be[b], 0, 0)),
        ],
        out_specs=pl.BlockSpec((M, D_MODEL), lambda b, be: (b, 0)),
    )
    return pl.pallas_call(
        _ffn_body,
        grid_spec=grid_spec,
        out_shape=jax.ShapeDtypeStruct((PAD_R, D_MODEL), jnp.float32),
    )(be, xs, scale3, w1, b1, w2, b2)


def _ffn_shared_body(x_ref, w1_ref, b1_ref, w2_ref, b2_ref, ys_ref):
    h = jnp.dot(x_ref[...], w1_ref[...], preferred_element_type=jnp.float32)
    h = h + b1_ref[...]
    h = 0.5 * h * (1.0 + lax.erf(h * 0.7071067811865476))
    y = jnp.dot(h, w2_ref[...], preferred_element_type=jnp.float32)
    ys_ref[...] = y + b2_ref[...]


def _ffn_shared(x2d, sw1, sb1, sw2, sb2):
    return pl.pallas_call(
        _ffn_shared_body,
        grid=(NBLK_S,),
        in_specs=[
            pl.BlockSpec((M, D_MODEL), lambda b: (b, 0)),
            pl.BlockSpec((D_MODEL, D_FF), lambda b: (0, 0)),
            pl.BlockSpec((1, D_FF), lambda b: (0, 0)),
            pl.BlockSpec((D_FF, D_MODEL), lambda b: (0, 0)),
            pl.BlockSpec((1, D_MODEL), lambda b: (0, 0)),
        ],
        out_specs=pl.BlockSpec((M, D_MODEL), lambda b: (b, 0)),
        out_shape=jax.ShapeDtypeStruct((SEQ, D_MODEL), jnp.float32),
    )(x2d, sw1, sb1, sw2, sb2)


_TPW = SEQ // _NW
_CCH = 32


def _combine_body(ysr_hbm, yss_hbm, pos_hbm, out_hbm,
                  idx0_v, idx1_v, r0_v, r1_v, rs_v, sem):
    wid = lax.axis_index("s") * _NC + lax.axis_index("c")
    base = wid * _TPW
    for c in range(_TPW // _CCH):
        t0 = base + c * _CCH
        pltpu.sync_copy(pos_hbm.at[0, pl.ds(t0, _CCH)], idx0_v)
        pltpu.sync_copy(pos_hbm.at[1, pl.ds(t0, _CCH)], idx1_v)
        cp0 = pltpu.async_copy(ysr_hbm.at[idx0_v], r0_v, sem)
        cp1 = pltpu.async_copy(ysr_hbm.at[idx1_v], r1_v, sem)
        cp2 = pltpu.async_copy(yss_hbm.at[pl.ds(t0, _CCH)], rs_v, sem)
        cp0.wait()
        cp1.wait()
        cp2.wait()

        @plsc.parallel_loop(0, _CCH * (D_MODEL // _L), unroll=8)
        def add_step(j):
            r = j // (D_MODEL // _L)
            col = (j % (D_MODEL // _L)) * _L
            acc = (r0_v[r, pl.ds(col, _L)] + r1_v[r, pl.ds(col, _L)]
                   + rs_v[r, pl.ds(col, _L)])
            rs_v[r, pl.ds(col, _L)] = acc
        pltpu.sync_copy(rs_v, out_hbm.at[pl.ds(t0, _CCH)])


def _combine(ys_r, ys_s, pos):
    mesh = plsc.VectorSubcoreMesh(core_axis_name="c", subcore_axis_name="s")
    f = pl.kernel(
        _combine_body,
        compiler_params=pltpu.CompilerParams(needs_layout_passes=False),
        out_type=jax.ShapeDtypeStruct((SEQ, D_MODEL), jnp.float32),
        mesh=mesh,
        scratch_types=[
            pltpu.VMEM((_CCH,), jnp.int32),
            pltpu.VMEM((_CCH,), jnp.int32),
            pltpu.VMEM((_CCH, D_MODEL), jnp.float32),
            pltpu.VMEM((_CCH, D_MODEL), jnp.float32),
            pltpu.VMEM((_CCH, D_MODEL), jnp.float32),
            pltpu.SemaphoreType.DMA,
        ],
    )
    return f(ys_r, ys_s, pos)


def kernel(x, gate_w, gate_b, sw1, sb1, sw2, sb2, rw1, rb1, rw2, rb2):
    x2d = x.reshape(SEQ, D_MODEL)
    pos, w, be = _router(x2d, gate_w, gate_b)
    ys_s = _ffn_shared(x2d, sw1[0], sb1, sw2[0], sb2)
    xsi, scale = _dispatch(pos, w, x2d)
    xs = lax.bitcast_convert_type(xsi, jnp.bfloat16).reshape(PAD_R, D_MODEL)
    b1 = rb1.reshape(N_EXPERT, 1, D_FF)
    b2 = rb2.reshape(N_EXPERT, 1, D_MODEL)
    ys_r = _ffn(be, xs, scale, rw1, b1, rw2, b2)
    out = _combine(ys_r, ys_s, pos)
    return out.reshape(x.shape)


# speedup vs baseline: 1.6486x; 1.6486x over previous
import functools

import jax
import jax.numpy as jnp
from jax import lax
from jax.experimental import pallas as pl
from jax.experimental.pallas import tpu as pltpu
from jax.experimental.pallas import tpu_sc as plsc

D_MODEL = 1024
D_FF = 2048
N_EXPERT = 8
TOP_K = 2
SEQ = 2048
M = 256
NBLK_R = (SEQ * TOP_K) // M + N_EXPERT
PAD_R = NBLK_R * M
NBLK_S = SEQ // M
NBLK = NBLK_R + NBLK_S
ROWS = PAD_R + SEQ


def _router_body(x_ref, gw_ref, gb_ref, pos_ref, w_ref, be_ref):
    x = x_ref[...]
    logits = jnp.dot(x, gw_ref[...], preferred_element_type=jnp.float32)
    logits = logits + gb_ref[...]
    mx = jnp.max(logits, axis=1, keepdims=True)
    ex = jnp.exp(logits - mx)
    probs = ex / jnp.sum(ex, axis=1, keepdims=True)

    iota_e = lax.broadcasted_iota(jnp.int32, (SEQ, N_EXPERT), 1)
    m1 = jnp.max(probs, axis=1, keepdims=True)
    i1 = jnp.min(jnp.where(probs == m1, iota_e, N_EXPERT), axis=1, keepdims=True)
    p2 = jnp.where(iota_e == i1, -1.0, probs)
    m2 = jnp.max(p2, axis=1, keepdims=True)
    i2 = jnp.min(jnp.where(p2 == m2, iota_e, N_EXPERT), axis=1, keepdims=True)

    sel1 = (iota_e == i1)
    sel2 = (iota_e == i2)
    cnt = (sel1 | sel2).astype(jnp.float32)

    ir = lax.broadcasted_iota(jnp.int32, (SEQ, SEQ), 0)
    ic = lax.broadcasted_iota(jnp.int32, (SEQ, SEQ), 1)
    lower = (ic < ir).astype(jnp.float32)
    csum = jnp.dot(lower, cnt, preferred_element_type=jnp.float32)

    counts = jnp.sum(cnt, axis=0, keepdims=True)
    padded = jnp.ceil(counts / M) * M
    ie_r = lax.broadcasted_iota(jnp.int32, (N_EXPERT, N_EXPERT), 0)
    ie_c = lax.broadcasted_iota(jnp.int32, (N_EXPERT, N_EXPERT), 1)
    tri = (ie_r < ie_c).astype(jnp.float32)
    off = jnp.dot(padded, tri, preferred_element_type=jnp.float32)
    end = off + padded

    slotf = off + csum
    pos1 = jnp.sum(jnp.where(sel1, slotf, 0.0), axis=1)
    pos2 = jnp.sum(jnp.where(sel2, slotf, 0.0), axis=1)
    pos_ref[0, :] = pos1.astype(jnp.int32)
    pos_ref[1, :] = pos2.astype(jnp.int32)
    w_ref[0, :] = jnp.sum(jnp.where(sel1, probs, 0.0), axis=1)
    w_ref[1, :] = jnp.sum(jnp.where(sel2, probs, 0.0), axis=1)

    bi = lax.broadcasted_iota(jnp.int32, (NBLK_R + 1, N_EXPERT), 0)
    ee = jnp.broadcast_to(end, (NBLK_R + 1, N_EXPERT))
    beh = jnp.sum((bi.astype(jnp.float32) * M >= ee).astype(jnp.int32), axis=1)
    beh = jnp.minimum(beh, N_EXPERT - 1)
    nused = (end[0, N_EXPERT - 1] / M).astype(jnp.int32)
    ii = lax.broadcasted_iota(jnp.int32, (NBLK_R + 1,), 0)
    be_ref[...] = jnp.where(ii < NBLK_R, beh, nused)


def _router(x2d, gate_w, gate_b):
    return pl.pallas_call(
        _router_body,
        out_shape=(
            jax.ShapeDtypeStruct((TOP_K, SEQ), jnp.int32),
            jax.ShapeDtypeStruct((TOP_K, SEQ), jnp.float32),
            jax.ShapeDtypeStruct((NBLK_R + 1,), jnp.int32),
        ),
    )(x2d, gate_w, gate_b.reshape(1, N_EXPERT))


_NC, _NS, _L = 2, 16, 16
_NW = _NC * _NS
_RPW = PAD_R // _NW
_GCH = 24
_NCH = _RPW // _GCH


_NBUF = 4


def _dispatch_body(pos_hbm, w_hbm, x_hbm, z_hbm, zf_hbm, xs_hbm, sc_hbm,
                   pos_v, w_v, tok_v, s_v, buf0, buf1, buf2, buf3, gsem, wsem):
    wid = lax.axis_index("s") * _NC + lax.axis_index("c")
    with jax.named_scope("disp_table"):
        pltpu.sync_copy(pos_hbm, pos_v)
        pltpu.sync_copy(w_hbm, w_v)
        pltpu.sync_copy(z_hbm, tok_v)
        pltpu.sync_copy(zf_hbm, s_v)

        lane = lax.iota(jnp.int32, _L)

        @plsc.parallel_loop(0, SEQ // _L, unroll=4)
        def scat_step(j):
            base = j * _L
            t = lane + base
            for k in range(TOP_K):
                idx = pos_v[k, pl.ds(base, _L)]
                plsc.store_scatter(tok_v, [idx], t)
                plsc.store_scatter(s_v, [idx], w_v[k, pl.ds(base, _L)])

    with jax.named_scope("disp_gather"):
        base_w = wid * _RPW
        bufs = (buf0, buf1, buf2, buf3)
        gcps = [None] * _NCH
        wcps = [None] * _NCH
        for c in range(_NCH):
            b = base_w + c * _GCH
            if c >= _NBUF:
                wcps[c - _NBUF].wait()
            gcps[c] = pltpu.async_copy(
                x_hbm.at[tok_v.at[pl.ds(b, _GCH)]], bufs[c % _NBUF], gsem)
            if c >= 1:
                gcps[c - 1].wait()
                wcps[c - 1] = pltpu.async_copy(
                    bufs[(c - 1) % _NBUF], xs_hbm.at[pl.ds(b - _GCH, _GCH)], wsem)
        gcps[_NCH - 1].wait()
        wcps[_NCH - 1] = pltpu.async_copy(
            bufs[(_NCH - 1) % _NBUF],
            xs_hbm.at[pl.ds(base_w + (_NCH - 1) * _GCH, _GCH)], wsem)
        for c in range(max(0, _NCH - _NBUF), _NCH):
            wcps[c].wait()
        pltpu.sync_copy(s_v.at[pl.ds(base_w, _RPW)], sc_hbm.at[pl.ds(base_w, _RPW)])


def _dispatch(pos, w, x2d):
    mesh = plsc.VectorSubcoreMesh(core_axis_name="c", subcore_axis_name="s")
    f = pl.kernel(
        _dispatch_body,
        compiler_params=pltpu.CompilerParams(needs_layout_passes=False),
        out_type=(
            jax.ShapeDtypeStruct((PAD_R, D_MODEL), jnp.float32),
            jax.ShapeDtypeStruct((PAD_R,), jnp.float32),
        ),
        mesh=mesh,
        scratch_types=[
            pltpu.VMEM((TOP_K, SEQ), jnp.int32),
            pltpu.VMEM((TOP_K, SEQ), jnp.float32),
            pltpu.VMEM((PAD_R,), jnp.int32),
            pltpu.VMEM((PAD_R,), jnp.float32),
            pltpu.VMEM((_GCH, D_MODEL), jnp.float32),
            pltpu.VMEM((_GCH, D_MODEL), jnp.float32),
            pltpu.VMEM((_GCH, D_MODEL), jnp.float32),
            pltpu.VMEM((_GCH, D_MODEL), jnp.float32),
            pltpu.SemaphoreType.DMA,
            pltpu.SemaphoreType.DMA,
        ],
    )
    return f(pos, w, x2d, jnp.zeros((PAD_R,), jnp.int32),
             jnp.zeros((PAD_R,), jnp.float32))


def _ffn_body(be_ref, xs_ref, sc_ref, w1_ref, b1_ref, w2_ref, b2_ref, ys_ref):
    @pl.when(pl.program_id(0) < be_ref[NBLK_R])
    def _():
        xb = xs_ref[...]
        h = jnp.dot(xb, w1_ref[0], preferred_element_type=jnp.float32)
        h = h + b1_ref[0]
        h = 0.5 * h * (1.0 + lax.erf(h * 0.7071067811865476))
        s = sc_ref[0, 0, :][:, None]
        h = h * s
        y = jnp.dot(h, w2_ref[0], preferred_element_type=jnp.float32)
        ys_ref[...] = y + s * b2_ref[0]


def _ffn(be, xs, scale, w1, b1, w2, b2):
    scale3 = scale.reshape(NBLK_R, 1, M)
    grid_spec = pltpu.PrefetchScalarGridSpec(
        num_scalar_prefetch=1,
        grid=(NBLK_R,),
        in_specs=[
            pl.BlockSpec((M, D_MODEL), lambda b, be: (b, 0)),
            pl.BlockSpec((1, 1, M), lambda b, be: (b, 0, 0)),
            pl.BlockSpec((1, D_MODEL, D_FF), lambda b, be: (be[b], 0, 0)),
            pl.BlockSpec((1, 1, D_FF), lambda b, be: (be[b], 0, 0)),
            pl.BlockSpec((1, D_FF, D_MODEL), lambda b, be: (be[b], 0, 0)),
            pl.BlockSpec((1, 1, D_MODEL), lambda b, be: (be[b], 0, 0)),
        ],
        out_specs=pl.BlockSpec((M, D_MODEL), lambda b, be: (b, 0)),
    )
    return pl.pallas_call(
        _ffn_body,
        grid_spec=grid_spec,
        out_shape=jax.ShapeDtypeStruct((PAD_R, D_MODEL), jnp.float32),
    )(be, xs, scale3, w1, b1, w2, b2)


def _ffn_shared_body(x_ref, w1_ref, b1_ref, w2_ref, b2_ref, ys_ref):
    h = jnp.dot(x_ref[...], w1_ref[...], preferred_element_type=jnp.float32)
    h = h + b1_ref[...]
    h = 0.5 * h * (1.0 + lax.erf(h * 0.7071067811865476))
    y = jnp.dot(h, w2_ref[...], preferred_element_type=jnp.float32)
    ys_ref[...] = y + b2_ref[...]


def _ffn_shared(x2d, sw1, sb1, sw2, sb2):
    return pl.pallas_call(
        _ffn_shared_body,
        grid=(NBLK_S,),
        in_specs=[
            pl.BlockSpec((M, D_MODEL), lambda b: (b, 0)),
            pl.BlockSpec((D_MODEL, D_FF), lambda b: (0, 0)),
            pl.BlockSpec((1, D_FF), lambda b: (0, 0)),
            pl.BlockSpec((D_FF, D_MODEL), lambda b: (0, 0)),
            pl.BlockSpec((1, D_MODEL), lambda b: (0, 0)),
        ],
        out_specs=pl.BlockSpec((M, D_MODEL), lambda b: (b, 0)),
        out_shape=jax.ShapeDtypeStruct((SEQ, D_MODEL), jnp.float32),
    )(x2d, sw1, sb1, sw2, sb2)


_TPW = SEQ // _NW
_CCH = 16
_NCHC = _TPW // _CCH


def _combine_body(ysr_hbm, yss_hbm, pos_hbm, out_hbm,
                  i0a, i1a, r0a, r1a, rsa, i0b, i1b, r0b, r1b, rsb, sem):
    wid = lax.axis_index("s") * _NC + lax.axis_index("c")
    base = wid * _TPW
    sets = ((i0a, i1a, r0a, r1a, rsa), (i0b, i1b, r0b, r1b, rsb))
    cps = [None] * _NCHC
    for c in range(_NCHC + 1):
        if c < _NCHC:
            i0, i1, r0, r1, rs = sets[c % 2]
            t0 = base + c * _CCH
            pltpu.sync_copy(pos_hbm.at[0, pl.ds(t0, _CCH)], i0)
            pltpu.sync_copy(pos_hbm.at[1, pl.ds(t0, _CCH)], i1)
            cps[c] = (pltpu.async_copy(ysr_hbm.at[i0], r0, sem),
                      pltpu.async_copy(ysr_hbm.at[i1], r1, sem),
                      pltpu.async_copy(yss_hbm.at[pl.ds(t0, _CCH)], rs, sem))
        if c >= 1:
            _, _, r0, r1, rs = sets[(c - 1) % 2]
            tp = base + (c - 1) * _CCH
            for cp in cps[c - 1]:
                cp.wait()

            @plsc.parallel_loop(0, _CCH * (D_MODEL // _L), unroll=8)
            def add_step(j):
                r = j // (D_MODEL // _L)
                col = (j % (D_MODEL // _L)) * _L
                acc = (r0[r, pl.ds(col, _L)] + r1[r, pl.ds(col, _L)]
                       + rs[r, pl.ds(col, _L)])
                rs[r, pl.ds(col, _L)] = acc

            pltpu.sync_copy(rs, out_hbm.at[pl.ds(tp, _CCH)])


def _combine(ys_r, ys_s, pos):
    mesh = plsc.VectorSubcoreMesh(core_axis_name="c", subcore_axis_name="s")
    f = pl.kernel(
        _combine_body,
        compiler_params=pltpu.CompilerParams(needs_layout_passes=False),
        out_type=jax.ShapeDtypeStruct((SEQ, D_MODEL), jnp.float32),
        mesh=mesh,
        scratch_types=[
            pltpu.VMEM((_CCH,), jnp.int32),
            pltpu.VMEM((_CCH,), jnp.int32),
            pltpu.VMEM((_CCH, D_MODEL), jnp.float32),
            pltpu.VMEM((_CCH, D_MODEL), jnp.float32),
            pltpu.VMEM((_CCH, D_MODEL), jnp.float32),
            pltpu.VMEM((_CCH,), jnp.int32),
            pltpu.VMEM((_CCH,), jnp.int32),
            pltpu.VMEM((_CCH, D_MODEL), jnp.float32),
            pltpu.VMEM((_CCH, D_MODEL), jnp.float32),
            pltpu.VMEM((_CCH, D_MODEL), jnp.float32),
            pltpu.SemaphoreType.DMA,
        ],
    )
    return f(ys_r, ys_s, pos)


def kernel(x, gate_w, gate_b, sw1, sb1, sw2, sb2, rw1, rb1, rw2, rb2):
    x2d = x.reshape(SEQ, D_MODEL)
    pos, w, be = _router(x2d, gate_w, gate_b)
    ys_s = _ffn_shared(x2d, sw1[0], sb1, sw2[0], sb2)
    xs, scale = _dispatch(pos, w, x2d)
    b1 = rb1.reshape(N_EXPERT, 1, D_FF)
    b2 = rb2.reshape(N_EXPERT, 1, D_MODEL)
    ys_r = _ffn(be, xs, scale, rw1, b1, rw2, b2)
    out = _combine(ys_r, ys_s, pos)
    return out.reshape(x.shape)


# speedup vs baseline: 1.9692x; 1.1945x over previous
import functools

import jax
import jax.numpy as jnp
from jax import lax
from jax.experimental import pallas as pl
from jax.experimental.pallas import tpu as pltpu
from jax.experimental.pallas import tpu_sc as plsc

D_MODEL = 1024
D_FF = 2048
N_EXPERT = 8
TOP_K = 2
SEQ = 2048
M = 256
NBLK_R = (SEQ * TOP_K) // M + N_EXPERT
PAD_R = NBLK_R * M
NBLK_S = SEQ // M
NBLK = NBLK_R + NBLK_S
_BE_LEN = 32
ROWS = PAD_R + SEQ


def _router_body(x_ref, gw_ref, gb_ref, pos_ref, w_ref, be_ref):
    x = x_ref[...]
    logits = jnp.dot(x, gw_ref[...], preferred_element_type=jnp.float32)
    logits = logits + gb_ref[...]
    mx = jnp.max(logits, axis=1, keepdims=True)
    ex = jnp.exp(logits - mx)
    probs = ex / jnp.sum(ex, axis=1, keepdims=True)

    iota_e = lax.broadcasted_iota(jnp.int32, (SEQ, N_EXPERT), 1)
    m1 = jnp.max(probs, axis=1, keepdims=True)
    i1 = jnp.min(jnp.where(probs == m1, iota_e, N_EXPERT), axis=1, keepdims=True)
    p2 = jnp.where(iota_e == i1, -1.0, probs)
    m2 = jnp.max(p2, axis=1, keepdims=True)
    i2 = jnp.min(jnp.where(p2 == m2, iota_e, N_EXPERT), axis=1, keepdims=True)

    sel1 = (iota_e == i1)
    sel2 = (iota_e == i2)
    cnt = (sel1 | sel2).astype(jnp.float32)

    ir = lax.broadcasted_iota(jnp.int32, (SEQ, SEQ), 0)
    ic = lax.broadcasted_iota(jnp.int32, (SEQ, SEQ), 1)
    lower = (ic < ir).astype(jnp.float32)
    csum = jnp.dot(lower, cnt, preferred_element_type=jnp.float32)

    counts = jnp.sum(cnt, axis=0, keepdims=True)
    padded = jnp.ceil(counts / M) * M
    ie_r = lax.broadcasted_iota(jnp.int32, (N_EXPERT, N_EXPERT), 0)
    ie_c = lax.broadcasted_iota(jnp.int32, (N_EXPERT, N_EXPERT), 1)
    tri = (ie_r < ie_c).astype(jnp.float32)
    off = jnp.dot(padded, tri, preferred_element_type=jnp.float32)
    end = off + padded

    slotf = off + csum
    pos1 = jnp.sum(jnp.where(sel1, slotf, 0.0), axis=1)
    pos2 = jnp.sum(jnp.where(sel2, slotf, 0.0), axis=1)
    pos_ref[0, :] = pos1.astype(jnp.int32)
    pos_ref[1, :] = pos2.astype(jnp.int32)
    w_ref[0, :] = jnp.sum(jnp.where(sel1, probs, 0.0), axis=1)
    w_ref[1, :] = jnp.sum(jnp.where(sel2, probs, 0.0), axis=1)

    bi = lax.broadcasted_iota(jnp.int32, (_BE_LEN, N_EXPERT), 0)
    ee = jnp.broadcast_to(end, (_BE_LEN, N_EXPERT))
    beh = jnp.sum((bi.astype(jnp.float32) * M >= ee).astype(jnp.int32), axis=1)
    beh = jnp.minimum(beh, N_EXPERT - 1)
    nused = (end[0, N_EXPERT - 1] / M).astype(jnp.int32)
    ii = lax.broadcasted_iota(jnp.int32, (_BE_LEN,), 0)
    be_ref[...] = jnp.where(ii < NBLK_R, beh, nused)


def _router(x2d, gate_w, gate_b):
    return pl.pallas_call(
        _router_body,
        out_shape=(
            jax.ShapeDtypeStruct((TOP_K, SEQ), jnp.int32),
            jax.ShapeDtypeStruct((TOP_K, SEQ), jnp.float32),
            jax.ShapeDtypeStruct((_BE_LEN,), jnp.int32),
        ),
    )(x2d, gate_w, gate_b.reshape(1, N_EXPERT))


_NC, _NS, _L = 2, 16, 16
_NW = _NC * _NS
_RPW = PAD_R // _NW
_GCH = 24
_NCH = _RPW // _GCH


_NBUF = 4


def _dispatch_body(pos_hbm, w_hbm, be_hbm, x_hbm, z_hbm, zf_hbm, xs_hbm, sc_hbm,
                   pos_v, w_v, tok_v, s_v, nu_v, buf0, buf1, buf2, buf3,
                   gsem, wsem):
    wid = lax.axis_index("s") * _NC + lax.axis_index("c")
    with jax.named_scope("disp_table"):
        pltpu.sync_copy(be_hbm.at[pl.ds(_BE_LEN - _L, _L)], nu_v)
        pltpu.sync_copy(pos_hbm, pos_v)
        pltpu.sync_copy(w_hbm, w_v)
        pltpu.sync_copy(z_hbm, tok_v)
        pltpu.sync_copy(zf_hbm, s_v)

        lane = lax.iota(jnp.int32, _L)

        @plsc.parallel_loop(0, SEQ // _L, unroll=4)
        def scat_step(j):
            base = j * _L
            t = lane + base
            for k in range(TOP_K):
                idx = pos_v[k, pl.ds(base, _L)]
                plsc.store_scatter(tok_v, [idx], t)
                plsc.store_scatter(s_v, [idx], w_v[k, pl.ds(base, _L)])

    with jax.named_scope("disp_gather"):
        base_w = wid * _RPW

        nused = nu_v[pl.ds(0, _L)][NBLK_R - (_BE_LEN - _L)]

        @pl.when(base_w < nused * M)
        def _():
            bufs = (buf0, buf1, buf2, buf3)
            gcps = [None] * _NCH
            wcps = [None] * _NCH
            for c in range(_NCH):
                b = base_w + c * _GCH
                if c >= _NBUF:
                    wcps[c - _NBUF].wait()
                gcps[c] = pltpu.async_copy(
                    x_hbm.at[tok_v.at[pl.ds(b, _GCH)]], bufs[c % _NBUF], gsem)
                if c >= 1:
                    gcps[c - 1].wait()
                    wcps[c - 1] = pltpu.async_copy(
                        bufs[(c - 1) % _NBUF], xs_hbm.at[pl.ds(b - _GCH, _GCH)],
                        wsem)
            gcps[_NCH - 1].wait()
            wcps[_NCH - 1] = pltpu.async_copy(
                bufs[(_NCH - 1) % _NBUF],
                xs_hbm.at[pl.ds(base_w + (_NCH - 1) * _GCH, _GCH)], wsem)
            for c in range(max(0, _NCH - _NBUF), _NCH):
                wcps[c].wait()
            pltpu.sync_copy(s_v.at[pl.ds(base_w, _RPW)],
                            sc_hbm.at[pl.ds(base_w, _RPW)])


def _dispatch(pos, w, be, x2d):
    mesh = plsc.VectorSubcoreMesh(core_axis_name="c", subcore_axis_name="s")
    f = pl.kernel(
        _dispatch_body,
        compiler_params=pltpu.CompilerParams(needs_layout_passes=False),
        out_type=(
            jax.ShapeDtypeStruct((PAD_R, D_MODEL), jnp.float32),
            jax.ShapeDtypeStruct((PAD_R,), jnp.float32),
        ),
        mesh=mesh,
        scratch_types=[
            pltpu.VMEM((TOP_K, SEQ), jnp.int32),
            pltpu.VMEM((TOP_K, SEQ), jnp.float32),
            pltpu.VMEM((PAD_R,), jnp.int32),
            pltpu.VMEM((PAD_R,), jnp.float32),
            pltpu.VMEM((_L,), jnp.int32),
            pltpu.VMEM((_GCH, D_MODEL), jnp.float32),
            pltpu.VMEM((_GCH, D_MODEL), jnp.float32),
            pltpu.VMEM((_GCH, D_MODEL), jnp.float32),
            pltpu.VMEM((_GCH, D_MODEL), jnp.float32),
            pltpu.SemaphoreType.DMA,
            pltpu.SemaphoreType.DMA,
        ],
    )
    return f(pos, w, be, x2d, jnp.zeros((PAD_R,), jnp.int32),
             jnp.zeros((PAD_R,), jnp.float32))


def _ffn_body(be_ref, xs_ref, sc_ref, w1_ref, b1_ref, w2_ref, b2_ref, ys_ref):
    @pl.when(pl.program_id(0) < be_ref[NBLK_R])
    def _():
        xb = xs_ref[...]
        h = jnp.dot(xb, w1_ref[0], preferred_element_type=jnp.float32)
        h = h + b1_ref[0]
        h = 0.5 * h * (1.0 + lax.erf(h * 0.7071067811865476))
        s = sc_ref[0, 0, :][:, None]
        h = h * s
        y = jnp.dot(h, w2_ref[0], preferred_element_type=jnp.float32)
        ys_ref[...] = y + s * b2_ref[0]


def _ffn(be, xs, scale, w1, b1, w2, b2):
    scale3 = scale.reshape(NBLK_R, 1, M)
    grid_spec = pltpu.PrefetchScalarGridSpec(
        num_scalar_prefetch=1,
        grid=(NBLK_R,),
        in_specs=[
            pl.BlockSpec((M, D_MODEL), lambda b, be: (b, 0)),
            pl.BlockSpec((1, 1, M), lambda b, be: (b, 0, 0)),
            pl.BlockSpec((1, D_MODEL, D_FF), lambda b, be: (be[b], 0, 0)),
            pl.BlockSpec((1, 1, D_FF), lambda b, be: (be[b], 0, 0)),
            pl.BlockSpec((1, D_FF, D_MODEL), lambda b, be: (be[b], 0, 0)),
            pl.BlockSpec((1, 1, D_MODEL), lambda b, be: (be[b], 0, 0)),
        ],
        out_specs=pl.BlockSpec((M, D_MODEL), lambda b, be: (b, 0)),
    )
    return pl.pallas_call(
        _ffn_body,
        grid_spec=grid_spec,
        out_shape=jax.ShapeDtypeStruct((PAD_R, D_MODEL), jnp.float32),
    )(be, xs, scale3, w1, b1, w2, b2)


def _ffn_shared_body(x_ref, w1_ref, b1_ref, w2_ref, b2_ref, ys_ref):
    h = jnp.dot(x_ref[...], w1_ref[...], preferred_element_type=jnp.float32)
    h = h + b1_ref[...]
    h = 0.5 * h * (1.0 + lax.erf(h * 0.7071067811865476))
    y = jnp.dot(h, w2_ref[...], preferred_element_type=jnp.float32)
    ys_ref[...] = y + b2_ref[...]


def _ffn_shared(x2d, sw1, sb1, sw2, sb2):
    return pl.pallas_call(
        _ffn_shared_body,
        grid=(NBLK_S,),
        in_specs=[
            pl.BlockSpec((M, D_MODEL), lambda b: (b, 0)),
            pl.BlockSpec((D_MODEL, D_FF), lambda b: (0, 0)),
            pl.BlockSpec((1, D_FF), lambda b: (0, 0)),
            pl.BlockSpec((D_FF, D_MODEL), lambda b: (0, 0)),
            pl.BlockSpec((1, D_MODEL), lambda b: (0, 0)),
        ],
        out_specs=pl.BlockSpec((M, D_MODEL), lambda b: (b, 0)),
        out_shape=jax.ShapeDtypeStruct((SEQ, D_MODEL), jnp.float32),
    )(x2d, sw1, sb1, sw2, sb2)


_TPW = SEQ // _NW
_CCH = 16
_NCHC = _TPW // _CCH


def _combine_body(ysr_hbm, yss_hbm, pos_hbm, out_hbm,
                  i0a, i1a, r0a, r1a, rsa, i0b, i1b, r0b, r1b, rsb, sem):
    wid = lax.axis_index("s") * _NC + lax.axis_index("c")
    base = wid * _TPW
    sets = ((i0a, i1a, r0a, r1a, rsa), (i0b, i1b, r0b, r1b, rsb))
    cps = [None] * _NCHC
    for c in range(_NCHC + 1):
        if c < _NCHC:
            i0, i1, r0, r1, rs = sets[c % 2]
            t0 = base + c * _CCH
            pltpu.sync_copy(pos_hbm.at[0, pl.ds(t0, _CCH)], i0)
            pltpu.sync_copy(pos_hbm.at[1, pl.ds(t0, _CCH)], i1)
            cps[c] = (pltpu.async_copy(ysr_hbm.at[i0], r0, sem),
                      pltpu.async_copy(ysr_hbm.at[i1], r1, sem),
                      pltpu.async_copy(yss_hbm.at[pl.ds(t0, _CCH)], rs, sem))
        if c >= 1:
            _, _, r0, r1, rs = sets[(c - 1) % 2]
            tp = base + (c - 1) * _CCH
            for cp in cps[c - 1]:
                cp.wait()

            @plsc.parallel_loop(0, _CCH * (D_MODEL // _L), unroll=8)
            def add_step(j):
                r = j // (D_MODEL // _L)
                col = (j % (D_MODEL // _L)) * _L
                acc = (r0[r, pl.ds(col, _L)] + r1[r, pl.ds(col, _L)]
                       + rs[r, pl.ds(col, _L)])
                rs[r, pl.ds(col, _L)] = acc

            pltpu.sync_copy(rs, out_hbm.at[pl.ds(tp, _CCH)])


def _combine(ys_r, ys_s, pos):
    mesh = plsc.VectorSubcoreMesh(core_axis_name="c", subcore_axis_name="s")
    f = pl.kernel(
        _combine_body,
        compiler_params=pltpu.CompilerParams(needs_layout_passes=False),
        out_type=jax.ShapeDtypeStruct((SEQ, D_MODEL), jnp.float32),
        mesh=mesh,
        scratch_types=[
            pltpu.VMEM((_CCH,), jnp.int32),
            pltpu.VMEM((_CCH,), jnp.int32),
            pltpu.VMEM((_CCH, D_MODEL), jnp.float32),
            pltpu.VMEM((_CCH, D_MODEL), jnp.float32),
            pltpu.VMEM((_CCH, D_MODEL), jnp.float32),
            pltpu.VMEM((_CCH,), jnp.int32),
            pltpu.VMEM((_CCH,), jnp.int32),
            pltpu.VMEM((_CCH, D_MODEL), jnp.float32),
            pltpu.VMEM((_CCH, D_MODEL), jnp.float32),
            pltpu.VMEM((_CCH, D_MODEL), jnp.float32),
            pltpu.SemaphoreType.DMA,
        ],
    )
    return f(ys_r, ys_s, pos)


def kernel(x, gate_w, gate_b, sw1, sb1, sw2, sb2, rw1, rb1, rw2, rb2):
    x2d = x.reshape(SEQ, D_MODEL)
    pos, w, be = _router(x2d, gate_w, gate_b)
    ys_s = _ffn_shared(x2d, sw1[0], sb1, sw2[0], sb2)
    xs, scale = _dispatch(pos, w, be, x2d)
    b1 = rb1.reshape(N_EXPERT, 1, D_FF)
    b2 = rb2.reshape(N_EXPERT, 1, D_MODEL)
    ys_r = _ffn(be, xs, scale, rw1, b1, rw2, b2)
    out = _combine(ys_r, ys_s, pos)
    return out.reshape(x.shape)
